# baseline (speedup 1.0000x reference)
import functools

import jax
import jax.numpy as jnp
from jax import lax
from jax.experimental import pallas as pl
from jax.experimental.pallas import tpu as pltpu
from jax.experimental.pallas import tpu_sc as plsc

N = 10000
E = 320000
GR = 64
D = 128
NEG = -3.0e38
CHUNK = 6400
L = 16


def _take(v, idx):
  return jnp.take_along_axis(v, idx, axis=0, mode="promise_in_bounds")


def _segmax_body(pq_hbm, src_hbm, dst_hbm, out_hbm, q0, q1, a0, a1, es, ed, tmp):
  c = lax.axis_index("c")
  s = lax.axis_index("s")
  w = s * 2 + c
  f0 = 2 * w

  pltpu.sync_copy(pq_hbm.at[GR + f0], q0)
  pltpu.sync_copy(pq_hbm.at[GR + f0 + 1], q1)

  neg = jnp.full((L,), NEG, jnp.float32)

  def init(i, carry):
    a0[pl.ds(i * L, L)] = neg
    a1[pl.ds(i * L, L)] = neg
    return carry

  lax.fori_loop(0, N // L, init, 0)

  iota = lax.iota(jnp.int32, L)
  all_true = jnp.ones((L,), jnp.bool_)

  def chunk_body(ci, carry):
    pltpu.sync_copy(src_hbm.at[pl.ds(ci * CHUNK, CHUNK)], es)
    pltpu.sync_copy(dst_hbm.at[pl.ds(ci * CHUNK, CHUNK)], ed)

    def vec_body(k, carry2):
      sv = es[pl.ds(k * L, L)]
      dv = ed[pl.ds(k * L, L)]
      v0 = plsc.load_gather(q0, [sv])
      v1 = plsc.load_gather(q1, [sv])

      def w_cond(state):
        return jnp.any(state[0])

      def w_body(state):
        pending = state[0]
        plsc.store_scatter(tmp, [dv], iota, mask=pending)
        rd = plsc.load_gather(tmp, [dv], mask=pending)
        win = jnp.logical_and(rd == iota, pending)
        c0 = plsc.load_gather(a0, [dv], mask=win)
        c1 = plsc.load_gather(a1, [dv], mask=win)
        plsc.store_scatter(a0, [dv], jnp.maximum(c0, v0), mask=win)
        plsc.store_scatter(a1, [dv], jnp.maximum(c1, v1), mask=win)
        return (jnp.logical_and(pending, jnp.logical_not(win)),)

      lax.while_loop(w_cond, w_body, (all_true,))
      return carry2

    lax.fori_loop(0, CHUNK // L, vec_body, 0)
    return carry

  lax.fori_loop(0, E // CHUNK, chunk_body, 0)

  pltpu.sync_copy(a0, out_hbm.at[f0])
  pltpu.sync_copy(a1, out_hbm.at[f0 + 1])


_segmax = functools.partial(
    pl.kernel,
    mesh=plsc.VectorSubcoreMesh(core_axis_name="c", subcore_axis_name="s"),
    out_type=jax.ShapeDtypeStruct((GR, N), jnp.float32),
    scratch_types=[
        pltpu.VMEM((N,), jnp.float32),
        pltpu.VMEM((N,), jnp.float32),
        pltpu.VMEM((N,), jnp.float32),
        pltpu.VMEM((N,), jnp.float32),
        pltpu.VMEM((CHUNK,), jnp.int32),
        pltpu.VMEM((CHUNK,), jnp.int32),
        pltpu.VMEM((L,), jnp.int32),
    ],
    compiler_params=pltpu.CompilerParams(needs_layout_passes=False),
)(_segmax_body)


def _tc0_body(xT, WlT, bl, Wc, bc, h0T_o, pq_o):
  h0 = jnp.dot(WlT[...], xT[...], preferred_element_type=jnp.float32) + bl[...]
  h0T_o[...] = h0
  pq_o[...] = jnp.dot(Wc[...], h0, preferred_element_type=jnp.float32) + bc[...]


_tc0 = pl.pallas_call(
    _tc0_body,
    out_shape=[
        jax.ShapeDtypeStruct((GR, N), jnp.float32),
        jax.ShapeDtypeStruct((2 * GR, N), jnp.float32),
    ],
)


def _tcb_body(nparts, pq, mT, Wc, bc, *refs):
  hrefs = refs[:nparts]
  agg_o, pq_o = refs[nparts], refs[nparts + 1]
  m = mT[...]
  agg = jnp.where(m > -1.0e30, pq[0:GR, :] + m, 0.0)
  agg_o[...] = agg
  hcat = jnp.concatenate([h[...] for h in hrefs] + [agg], axis=0)
  pq_o[...] = jnp.dot(Wc[...], hcat, preferred_element_type=jnp.float32) + bc[...]


def _make_tcb(nparts):
  return pl.pallas_call(
      functools.partial(_tcb_body, nparts),
      out_shape=[
          jax.ShapeDtypeStruct((GR, N), jnp.float32),
          jax.ShapeDtypeStruct((2 * GR, N), jnp.float32),
      ],
  )


_tcb1 = _make_tcb(1)
_tcb2 = _make_tcb(2)


def _pool4(S):
  return jnp.max(S.reshape(GR // 4, 4, S.shape[-1]), axis=1)


def _tcf_body(h0T, a0T, a1T, pq, mT, out_o):
  m = mT[...]
  a2 = jnp.where(m > -1.0e30, pq[0:GR, :] + m, 0.0)
  out_o[...] = jnp.concatenate(
      [_pool4(h0T[...]), _pool4(a0T[...]), _pool4(a1T[...]), _pool4(a2)],
      axis=0,
  )


_tcf = pl.pallas_call(
    _tcf_body,
    out_shape=jax.ShapeDtypeStruct((GR, N), jnp.float32),
)


def kernel(x, edge_index, lin_x_W, lin_x_b, W0, b0, W1, b1, W2, b2):
  xT = x.T
  src = edge_index[0]
  dst = edge_index[1]

  Wcs, bcs = [], []
  for i, (W, b) in enumerate(((W0, b0), (W1, b1), (W2, b2))):
    cin = (i + 1) * GR
    Wt = W[:cin].T
    Wb = W[cin:].T
    Wcs.append(jnp.concatenate([Wt - Wb, Wb], axis=0))
    bcs.append(jnp.concatenate([b, jnp.zeros((GR,), jnp.float32)])[:, None])

  h0T, pq = _tc0(xT, lin_x_W.T, lin_x_b[:, None], Wcs[0], bcs[0])
  m0 = _segmax(pq, src, dst)
  agg0, pq = _tcb1(pq, m0, Wcs[1], bcs[1], h0T)
  m1 = _segmax(pq, src, dst)
  agg1, pq = _tcb2(pq, m1, Wcs[2], bcs[2], h0T, agg0)
  m2 = _segmax(pq, src, dst)
  outT = _tcf(h0T, agg0, agg1, pq, m2)
  return outT.T


# speedup vs baseline: 1.4339x; 1.4339x over previous
import functools

import jax
import jax.numpy as jnp
from jax import lax
from jax.experimental import pallas as pl
from jax.experimental.pallas import tpu as pltpu
from jax.experimental.pallas import tpu_sc as plsc

N = 10000
E = 320000
GR = 64
D = 128
NEG = -3.0e38
CHUNK = 4000
L = 16
FPW = 4
EH = E // 2


def _segmax_body(pq_hbm, src_hbm, dst_hbm, out_hbm, q0, q1, q2, q3, a0, a1,
                 a2, a3, es, ed, tmp):
  q = (q0, q1, q2, q3)
  a = (a0, a1, a2, a3)
  half = lax.axis_index("c")
  s = lax.axis_index("s")
  f0 = FPW * s

  for j in range(FPW):
    pltpu.sync_copy(pq_hbm.at[GR + f0 + j], q[j])

  neg = jnp.full((L,), NEG, jnp.float32)

  def init(i, carry):
    for j in range(FPW):
      a[j][pl.ds(i * L, L)] = neg
    return carry

  lax.fori_loop(0, N // L, init, 0)

  iota = lax.iota(jnp.int32, L)
  tru = jnp.ones((L,), jnp.bool_)

  def chunk_body(ci, carry):
    base = half * EH + ci * CHUNK
    pltpu.sync_copy(src_hbm.at[pl.ds(base, CHUNK)], es)
    pltpu.sync_copy(dst_hbm.at[pl.ds(base, CHUNK)], ed)

    def vec_body(k, viol):
      sv = es[pl.ds(k * L, L)]
      dv = ed[pl.ds(k * L, L)]
      vs = [plsc.load_gather(q[j], [sv]) for j in range(FPW)]
      plsc.store_scatter(tmp, [dv], iota)
      rd = plsc.load_gather(tmp, [dv])
      win = rd == iota
      for j in range(FPW):
        cj = plsc.load_gather(a[j], [dv])
        plsc.store_scatter(a[j], [dv], jnp.maximum(cj, vs[j]), mask=win)
      pend = jnp.logical_not(win)
      plsc.store_scatter(tmp, [dv], iota, mask=pend)
      rd2 = plsc.load_gather(tmp, [dv], mask=pend)
      win2 = jnp.logical_and(rd2 == iota, pend)
      for j in range(FPW):
        cj = plsc.load_gather(a[j], [dv], mask=win2)
        plsc.store_scatter(a[j], [dv], jnp.maximum(cj, vs[j]), mask=win2)
      return jnp.logical_or(viol, jnp.logical_and(pend, jnp.logical_not(win2)))

    viol = lax.fori_loop(0, CHUNK // L, vec_body, jnp.zeros((L,), jnp.bool_))

    @pl.when(jnp.any(viol))
    def _slow_redo():
      def vec_slow(k, carry2):
        sv = es[pl.ds(k * L, L)]
        dv = ed[pl.ds(k * L, L)]
        vs = [plsc.load_gather(q[j], [sv]) for j in range(FPW)]

        def w_cond(state):
          return jnp.any(state[0])

        def w_body(state):
          pending = state[0]
          plsc.store_scatter(tmp, [dv], iota, mask=pending)
          rdw = plsc.load_gather(tmp, [dv], mask=pending)
          w_ = jnp.logical_and(rdw == iota, pending)
          for j in range(FPW):
            cj = plsc.load_gather(a[j], [dv], mask=w_)
            plsc.store_scatter(a[j], [dv], jnp.maximum(cj, vs[j]), mask=w_)
          return (jnp.logical_and(pending, jnp.logical_not(w_)),)

        lax.while_loop(w_cond, w_body, (tru,))
        return carry2

      lax.fori_loop(0, CHUNK // L, vec_slow, 0)

    return carry

  lax.fori_loop(0, EH // CHUNK, chunk_body, 0)

  for j in range(FPW):
    pltpu.sync_copy(a[j], out_hbm.at[half, f0 + j])


_segmax = functools.partial(
    pl.kernel,
    mesh=plsc.VectorSubcoreMesh(core_axis_name="c", subcore_axis_name="s"),
    out_type=jax.ShapeDtypeStruct((2, GR, N), jnp.float32),
    scratch_types=[
        pltpu.VMEM((N,), jnp.float32),
        pltpu.VMEM((N,), jnp.float32),
        pltpu.VMEM((N,), jnp.float32),
        pltpu.VMEM((N,), jnp.float32),
        pltpu.VMEM((N,), jnp.float32),
        pltpu.VMEM((N,), jnp.float32),
        pltpu.VMEM((N,), jnp.float32),
        pltpu.VMEM((N,), jnp.float32),
        pltpu.VMEM((CHUNK,), jnp.int32),
        pltpu.VMEM((CHUNK,), jnp.int32),
        pltpu.VMEM((L,), jnp.int32),
    ],
    compiler_params=pltpu.CompilerParams(needs_layout_passes=False),
)(_segmax_body)


def _tc0_body(xT, WlT, bl, Wc, bc, h0T_o, pq_o):
  h0 = jnp.dot(WlT[...], xT[...], preferred_element_type=jnp.float32) + bl[...]
  h0T_o[...] = h0
  pq_o[...] = jnp.dot(Wc[...], h0, preferred_element_type=jnp.float32) + bc[...]


_tc0 = pl.pallas_call(
    _tc0_body,
    out_shape=[
        jax.ShapeDtypeStruct((GR, N), jnp.float32),
        jax.ShapeDtypeStruct((2 * GR, N), jnp.float32),
    ],
)


def _tcb_body(nparts, pq, mT, Wc, bc, *refs):
  hrefs = refs[:nparts]
  agg_o, pq_o = refs[nparts], refs[nparts + 1]
  m = jnp.maximum(mT[0], mT[1])
  agg = jnp.where(m > -1.0e30, pq[0:GR, :] + m, 0.0)
  agg_o[...] = agg
  hcat = jnp.concatenate([h[...] for h in hrefs] + [agg], axis=0)
  pq_o[...] = jnp.dot(Wc[...], hcat, preferred_element_type=jnp.float32) + bc[...]


def _make_tcb(nparts):
  return pl.pallas_call(
      functools.partial(_tcb_body, nparts),
      out_shape=[
          jax.ShapeDtypeStruct((GR, N), jnp.float32),
          jax.ShapeDtypeStruct((2 * GR, N), jnp.float32),
      ],
  )


_tcb1 = _make_tcb(1)
_tcb2 = _make_tcb(2)


def _pool4(S):
  return jnp.max(S.reshape(GR // 4, 4, S.shape[-1]), axis=1)


def _tcf_body(h0T, a0T, a1T, pq, mT, out_o):
  m = jnp.maximum(mT[0], mT[1])
  a2 = jnp.where(m > -1.0e30, pq[0:GR, :] + m, 0.0)
  out_o[...] = jnp.concatenate(
      [_pool4(h0T[...]), _pool4(a0T[...]), _pool4(a1T[...]), _pool4(a2)],
      axis=0,
  )


_tcf = pl.pallas_call(
    _tcf_body,
    out_shape=jax.ShapeDtypeStruct((GR, N), jnp.float32),
)


def kernel(x, edge_index, lin_x_W, lin_x_b, W0, b0, W1, b1, W2, b2):
  xT = x.T
  src = edge_index[0]
  dst = edge_index[1]

  Wcs, bcs = [], []
  for i, (W, b) in enumerate(((W0, b0), (W1, b1), (W2, b2))):
    cin = (i + 1) * GR
    Wt = W[:cin].T
    Wb = W[cin:].T
    Wcs.append(jnp.concatenate([Wt - Wb, Wb], axis=0))
    bcs.append(jnp.concatenate([b, jnp.zeros((GR,), jnp.float32)])[:, None])

  h0T, pq = _tc0(xT, lin_x_W.T, lin_x_b[:, None], Wcs[0], bcs[0])
  m0 = _segmax(pq, src, dst)
  agg0, pq = _tcb1(pq, m0, Wcs[1], bcs[1], h0T)
  m1 = _segmax(pq, src, dst)
  agg1, pq = _tcb2(pq, m1, Wcs[2], bcs[2], h0T, agg0)
  m2 = _segmax(pq, src, dst)
  outT = _tcf(h0T, agg0, agg1, pq, m2)
  return outT.T


# speedup vs baseline: 1.4403x; 1.0045x over previous
import functools

import jax
import jax.numpy as jnp
from jax import lax
from jax.experimental import pallas as pl
from jax.experimental.pallas import tpu as pltpu
from jax.experimental.pallas import tpu_sc as plsc

N = 10000
E = 320000
GR = 64
D = 128
NEG = -3.0e38
CHUNK = 4000
L = 16
FPW = 4
EH = E // 2


def _segmax_body(pq_hbm, src_hbm, dst_hbm, out_hbm, q0, q1, q2, q3, a0, a1,
                 a2, a3, es, ed, tmp, tmp2):
  q = (q0, q1, q2, q3)
  a = (a0, a1, a2, a3)
  half = lax.axis_index("c")
  s = lax.axis_index("s")
  f0 = FPW * s

  for j in range(FPW):
    pltpu.sync_copy(pq_hbm.at[GR + f0 + j], q[j])

  neg = jnp.full((L,), NEG, jnp.float32)

  def init(i, carry):
    for j in range(FPW):
      a[j][pl.ds(i * L, L)] = neg
    return carry

  lax.fori_loop(0, N // L, init, 0)

  iota = lax.iota(jnp.int32, L)
  tru = jnp.ones((L,), jnp.bool_)

  def chunk_body(ci, carry):
    base = half * EH + ci * CHUNK
    pltpu.sync_copy(src_hbm.at[pl.ds(base, CHUNK)], es)
    pltpu.sync_copy(dst_hbm.at[pl.ds(base, CHUNK)], ed)

    def vec_one(k, viol, tmp_u):
      sv = es[pl.ds(k * L, L)]
      dv = ed[pl.ds(k * L, L)]
      vs = [plsc.load_gather(q[j], [sv]) for j in range(FPW)]
      plsc.store_scatter(tmp_u, [dv], iota)
      rd = plsc.load_gather(tmp_u, [dv])
      win = rd == iota
      for j in range(FPW):
        cj = plsc.load_gather(a[j], [dv])
        plsc.store_scatter(a[j], [dv], jnp.maximum(cj, vs[j]), mask=win)
      pend = jnp.logical_not(win)
      plsc.store_scatter(tmp_u, [dv], iota, mask=pend)
      rd2 = plsc.load_gather(tmp_u, [dv], mask=pend)
      win2 = jnp.logical_and(rd2 == iota, pend)
      for j in range(FPW):
        cj = plsc.load_gather(a[j], [dv], mask=win2)
        plsc.store_scatter(a[j], [dv], jnp.maximum(cj, vs[j]), mask=win2)
      return jnp.logical_or(viol, jnp.logical_and(pend, jnp.logical_not(win2)))

    def vec_body(k2, viol):
      viol = vec_one(2 * k2, viol, tmp)
      viol = vec_one(2 * k2 + 1, viol, tmp2)
      return viol

    viol = lax.fori_loop(0, CHUNK // L // 2, vec_body,
                         jnp.zeros((L,), jnp.bool_))

    @pl.when(jnp.any(viol))
    def _slow_redo():
      def vec_slow(k, carry2):
        sv = es[pl.ds(k * L, L)]
        dv = ed[pl.ds(k * L, L)]
        vs = [plsc.load_gather(q[j], [sv]) for j in range(FPW)]

        def w_cond(state):
          return jnp.any(state[0])

        def w_body(state):
          pending = state[0]
          plsc.store_scatter(tmp, [dv], iota, mask=pending)
          rdw = plsc.load_gather(tmp, [dv], mask=pending)
          w_ = jnp.logical_and(rdw == iota, pending)
          for j in range(FPW):
            cj = plsc.load_gather(a[j], [dv], mask=w_)
            plsc.store_scatter(a[j], [dv], jnp.maximum(cj, vs[j]), mask=w_)
          return (jnp.logical_and(pending, jnp.logical_not(w_)),)

        lax.while_loop(w_cond, w_body, (tru,))
        return carry2

      lax.fori_loop(0, CHUNK // L, vec_slow, 0)

    return carry

  lax.fori_loop(0, EH // CHUNK, chunk_body, 0)

  for j in range(FPW):
    pltpu.sync_copy(a[j], out_hbm.at[half, f0 + j])


_segmax = functools.partial(
    pl.kernel,
    mesh=plsc.VectorSubcoreMesh(core_axis_name="c", subcore_axis_name="s"),
    out_type=jax.ShapeDtypeStruct((2, GR, N), jnp.float32),
    scratch_types=[
        pltpu.VMEM((N,), jnp.float32),
        pltpu.VMEM((N,), jnp.float32),
        pltpu.VMEM((N,), jnp.float32),
        pltpu.VMEM((N,), jnp.float32),
        pltpu.VMEM((N,), jnp.float32),
        pltpu.VMEM((N,), jnp.float32),
        pltpu.VMEM((N,), jnp.float32),
        pltpu.VMEM((N,), jnp.float32),
        pltpu.VMEM((CHUNK,), jnp.int32),
        pltpu.VMEM((CHUNK,), jnp.int32),
        pltpu.VMEM((L,), jnp.int32),
        pltpu.VMEM((L,), jnp.int32),
    ],
    compiler_params=pltpu.CompilerParams(needs_layout_passes=False),
)(_segmax_body)


def _tc0_body(xT, WlT, bl, Wc, bc, h0T_o, pq_o):
  h0 = jnp.dot(WlT[...], xT[...], preferred_element_type=jnp.float32) + bl[...]
  h0T_o[...] = h0
  pq_o[...] = jnp.dot(Wc[...], h0, preferred_element_type=jnp.float32) + bc[...]


_tc0 = pl.pallas_call(
    _tc0_body,
    out_shape=[
        jax.ShapeDtypeStruct((GR, N), jnp.float32),
        jax.ShapeDtypeStruct((2 * GR, N), jnp.float32),
    ],
)


def _tcb_body(nparts, pq, mT, Wc, bc, *refs):
  hrefs = refs[:nparts]
  agg_o, pq_o = refs[nparts], refs[nparts + 1]
  m = jnp.maximum(mT[0], mT[1])
  agg = jnp.where(m > -1.0e30, pq[0:GR, :] + m, 0.0)
  agg_o[...] = agg
  hcat = jnp.concatenate([h[...] for h in hrefs] + [agg], axis=0)
  pq_o[...] = jnp.dot(Wc[...], hcat, preferred_element_type=jnp.float32) + bc[...]


def _make_tcb(nparts):
  return pl.pallas_call(
      functools.partial(_tcb_body, nparts),
      out_shape=[
          jax.ShapeDtypeStruct((GR, N), jnp.float32),
          jax.ShapeDtypeStruct((2 * GR, N), jnp.float32),
      ],
  )


_tcb1 = _make_tcb(1)
_tcb2 = _make_tcb(2)


def _pool4(S):
  return jnp.max(S.reshape(GR // 4, 4, S.shape[-1]), axis=1)


def _tcf_body(h0T, a0T, a1T, pq, mT, out_o):
  m = jnp.maximum(mT[0], mT[1])
  a2 = jnp.where(m > -1.0e30, pq[0:GR, :] + m, 0.0)
  out_o[...] = jnp.concatenate(
      [_pool4(h0T[...]), _pool4(a0T[...]), _pool4(a1T[...]), _pool4(a2)],
      axis=0,
  )


_tcf = pl.pallas_call(
    _tcf_body,
    out_shape=jax.ShapeDtypeStruct((GR, N), jnp.float32),
)


def kernel(x, edge_index, lin_x_W, lin_x_b, W0, b0, W1, b1, W2, b2):
  xT = x.T
  src = edge_index[0]
  dst = edge_index[1]

  Wcs, bcs = [], []
  for i, (W, b) in enumerate(((W0, b0), (W1, b1), (W2, b2))):
    cin = (i + 1) * GR
    Wt = W[:cin].T
    Wb = W[cin:].T
    Wcs.append(jnp.concatenate([Wt - Wb, Wb], axis=0))
    bcs.append(jnp.concatenate([b, jnp.zeros((GR,), jnp.float32)])[:, None])

  h0T, pq = _tc0(xT, lin_x_W.T, lin_x_b[:, None], Wcs[0], bcs[0])
  m0 = _segmax(pq, src, dst)
  agg0, pq = _tcb1(pq, m0, Wcs[1], bcs[1], h0T)
  m1 = _segmax(pq, src, dst)
  agg1, pq = _tcb2(pq, m1, Wcs[2], bcs[2], h0T, agg0)
  m2 = _segmax(pq, src, dst)
  outT = _tcf(h0T, agg0, agg1, pq, m2)
  return outT.T


# speedup vs baseline: 1.5900x; 1.1039x over previous
import functools

import jax
import jax.numpy as jnp
from jax import lax
from jax.experimental import pallas as pl
from jax.experimental.pallas import tpu as pltpu
from jax.experimental.pallas import tpu_sc as plsc

N = 10000
E = 320000
GR = 64
D = 128
NEG = -3.0e38
CHUNK = 4000
L = 16
FPW = 4
EH = E // 2


def _segmax_body(pq_hbm, src_hbm, dst_hbm, out_hbm, q0, q1, q2, q3, a0, a1,
                 a2, a3, es0, es1, ed0, ed1, tmp, tmp2, ss0, ss1, sd0, sd1):
  q = (q0, q1, q2, q3)
  a = (a0, a1, a2, a3)
  ebufs = ((es0, ed0, ss0, sd0), (es1, ed1, ss1, sd1))
  half = lax.axis_index("c")
  s = lax.axis_index("s")
  f0 = FPW * s

  for j in range(FPW):
    pltpu.sync_copy(pq_hbm.at[GR + f0 + j], q[j])

  neg = jnp.full((L,), NEG, jnp.float32)

  def init(i, carry):
    for j in range(FPW):
      a[j][pl.ds(i * L, L)] = neg
    return carry

  lax.fori_loop(0, N // L, init, 0)

  iota = lax.iota(jnp.int32, L)
  tru = jnp.ones((L,), jnp.bool_)

  nchunks = EH // CHUNK

  def edma_start(ci, buf):
    esb, edb, ss, sd = buf
    base = half * EH + ci * CHUNK
    pltpu.make_async_copy(src_hbm.at[pl.ds(base, CHUNK)], esb, ss).start()
    pltpu.make_async_copy(dst_hbm.at[pl.ds(base, CHUNK)], edb, sd).start()

  def edma_wait(ci, buf):
    esb, edb, ss, sd = buf
    base = half * EH + ci * CHUNK
    pltpu.make_async_copy(src_hbm.at[pl.ds(base, CHUNK)], esb, ss).wait()
    pltpu.make_async_copy(dst_hbm.at[pl.ds(base, CHUNK)], edb, sd).wait()

  def chunk_work(ci, es, ed):

    def vec_one(k, viol, tmp_u):
      sv = es[pl.ds(k * L, L)]
      dv = ed[pl.ds(k * L, L)]
      vs = [plsc.load_gather(q[j], [sv]) for j in range(FPW)]
      plsc.store_scatter(tmp_u, [dv], iota)
      rd = plsc.load_gather(tmp_u, [dv])
      win = rd == iota
      for j in range(FPW):
        cj = plsc.load_gather(a[j], [dv])
        plsc.store_scatter(a[j], [dv], jnp.maximum(cj, vs[j]), mask=win)
      pend = jnp.logical_not(win)
      plsc.store_scatter(tmp_u, [dv], iota, mask=pend)
      rd2 = plsc.load_gather(tmp_u, [dv], mask=pend)
      win2 = jnp.logical_and(rd2 == iota, pend)
      for j in range(FPW):
        cj = plsc.load_gather(a[j], [dv], mask=win2)
        plsc.store_scatter(a[j], [dv], jnp.maximum(cj, vs[j]), mask=win2)
      return jnp.logical_or(viol, jnp.logical_and(pend, jnp.logical_not(win2)))

    def vec_body(k2, viol):
      viol = vec_one(2 * k2, viol, tmp)
      viol = vec_one(2 * k2 + 1, viol, tmp2)
      return viol

    viol = lax.fori_loop(0, CHUNK // L // 2, vec_body,
                         jnp.zeros((L,), jnp.bool_))

    @pl.when(jnp.any(viol))
    def _slow_redo():
      def vec_slow(k, carry2):
        sv = es[pl.ds(k * L, L)]
        dv = ed[pl.ds(k * L, L)]
        vs = [plsc.load_gather(q[j], [sv]) for j in range(FPW)]

        def w_cond(state):
          return jnp.any(state[0])

        def w_body(state):
          pending = state[0]
          plsc.store_scatter(tmp, [dv], iota, mask=pending)
          rdw = plsc.load_gather(tmp, [dv], mask=pending)
          w_ = jnp.logical_and(rdw == iota, pending)
          for j in range(FPW):
            cj = plsc.load_gather(a[j], [dv], mask=w_)
            plsc.store_scatter(a[j], [dv], jnp.maximum(cj, vs[j]), mask=w_)
          return (jnp.logical_and(pending, jnp.logical_not(w_)),)

        lax.while_loop(w_cond, w_body, (tru,))
        return carry2

      lax.fori_loop(0, CHUNK // L, vec_slow, 0)

  def outer(ci2, carry):
    for par in range(2):
      ci = ci2 * 2 + par
      edma_wait(ci, ebufs[par])

      @pl.when(ci + 1 < nchunks)
      def _prefetch():
        edma_start(ci + 1, ebufs[1 - par])

      chunk_work(ci, ebufs[par][0], ebufs[par][1])
    return carry

  edma_start(0, ebufs[0])
  lax.fori_loop(0, nchunks // 2, outer, 0)

  for j in range(FPW):
    pltpu.sync_copy(a[j], out_hbm.at[half, f0 + j])


_segmax = functools.partial(
    pl.kernel,
    mesh=plsc.VectorSubcoreMesh(core_axis_name="c", subcore_axis_name="s"),
    out_type=jax.ShapeDtypeStruct((2, GR, N), jnp.float32),
    scratch_types=[
        pltpu.VMEM((N,), jnp.float32),
        pltpu.VMEM((N,), jnp.float32),
        pltpu.VMEM((N,), jnp.float32),
        pltpu.VMEM((N,), jnp.float32),
        pltpu.VMEM((N,), jnp.float32),
        pltpu.VMEM((N,), jnp.float32),
        pltpu.VMEM((N,), jnp.float32),
        pltpu.VMEM((N,), jnp.float32),
        pltpu.VMEM((CHUNK,), jnp.int32),
        pltpu.VMEM((CHUNK,), jnp.int32),
        pltpu.VMEM((CHUNK,), jnp.int32),
        pltpu.VMEM((CHUNK,), jnp.int32),
        pltpu.VMEM((L,), jnp.int32),
        pltpu.VMEM((L,), jnp.int32),
        pltpu.SemaphoreType.DMA,
        pltpu.SemaphoreType.DMA,
        pltpu.SemaphoreType.DMA,
        pltpu.SemaphoreType.DMA,
    ],
    compiler_params=pltpu.CompilerParams(needs_layout_passes=False),
)(_segmax_body)


def _tc0_body(xT, WlT, bl, Wc, bc, h0T_o, pq_o):
  h0 = jnp.dot(WlT[...], xT[...], preferred_element_type=jnp.float32) + bl[...]
  h0T_o[...] = h0
  pq_o[...] = jnp.dot(Wc[...], h0, preferred_element_type=jnp.float32) + bc[...]


_tc0 = pl.pallas_call(
    _tc0_body,
    out_shape=[
        jax.ShapeDtypeStruct((GR, N), jnp.float32),
        jax.ShapeDtypeStruct((2 * GR, N), jnp.float32),
    ],
)


def _tcb_body(nparts, pq, mT, Wc, bc, *refs):
  hrefs = refs[:nparts]
  agg_o, pq_o = refs[nparts], refs[nparts + 1]
  m = jnp.maximum(mT[0], mT[1])
  agg = jnp.where(m > -1.0e30, pq[0:GR, :] + m, 0.0)
  agg_o[...] = agg
  hcat = jnp.concatenate([h[...] for h in hrefs] + [agg], axis=0)
  pq_o[...] = jnp.dot(Wc[...], hcat, preferred_element_type=jnp.float32) + bc[...]


def _make_tcb(nparts):
  return pl.pallas_call(
      functools.partial(_tcb_body, nparts),
      out_shape=[
          jax.ShapeDtypeStruct((GR, N), jnp.float32),
          jax.ShapeDtypeStruct((2 * GR, N), jnp.float32),
      ],
  )


_tcb1 = _make_tcb(1)
_tcb2 = _make_tcb(2)


def _pool4(S):
  return jnp.max(S.reshape(GR // 4, 4, S.shape[-1]), axis=1)


def _tcf_body(h0T, a0T, a1T, pq, mT, out_o):
  m = jnp.maximum(mT[0], mT[1])
  a2 = jnp.where(m > -1.0e30, pq[0:GR, :] + m, 0.0)
  out_o[...] = jnp.concatenate(
      [_pool4(h0T[...]), _pool4(a0T[...]), _pool4(a1T[...]), _pool4(a2)],
      axis=0,
  )


_tcf = pl.pallas_call(
    _tcf_body,
    out_shape=jax.ShapeDtypeStruct((GR, N), jnp.float32),
)


def kernel(x, edge_index, lin_x_W, lin_x_b, W0, b0, W1, b1, W2, b2):
  xT = x.T
  src = edge_index[0]
  dst = edge_index[1]

  Wcs, bcs = [], []
  for i, (W, b) in enumerate(((W0, b0), (W1, b1), (W2, b2))):
    cin = (i + 1) * GR
    Wt = W[:cin].T
    Wb = W[cin:].T
    Wcs.append(jnp.concatenate([Wt - Wb, Wb], axis=0))
    bcs.append(jnp.concatenate([b, jnp.zeros((GR,), jnp.float32)])[:, None])

  h0T, pq = _tc0(xT, lin_x_W.T, lin_x_b[:, None], Wcs[0], bcs[0])
  m0 = _segmax(pq, src, dst)
  agg0, pq = _tcb1(pq, m0, Wcs[1], bcs[1], h0T)
  m1 = _segmax(pq, src, dst)
  agg1, pq = _tcb2(pq, m1, Wcs[2], bcs[2], h0T, agg0)
  m2 = _segmax(pq, src, dst)
  outT = _tcf(h0T, agg0, agg1, pq, m2)
  return outT.T


# speedup vs baseline: 2.7601x; 1.7360x over previous
import functools

import jax
import jax.numpy as jnp
from jax import lax
from jax.experimental import pallas as pl
from jax.experimental.pallas import tpu as pltpu
from jax.experimental.pallas import tpu_sc as plsc

N = 10000
E = 320000
GR = 64
D = 128
NEG = -3.0e38
CHUNK = 4000
L = 16
FPW = 4
EH = E // 2


def _segmax_body(pq_hbm, src_hbm, dst_hbm, out_hbm, q0, q1, q2, q3, a0, a1,
                 a2, a3, es0, es1, ed0, ed1, tmp, tmp2, ss0, ss1, sd0, sd1):
  q = (q0, q1, q2, q3)
  a = (a0, a1, a2, a3)
  ebufs = ((es0, ed0, ss0, sd0), (es1, ed1, ss1, sd1))
  half = lax.axis_index("c")
  s = lax.axis_index("s")
  f0 = FPW * s

  for j in range(FPW):
    pltpu.sync_copy(pq_hbm.at[GR + f0 + j], q[j])

  neg = jnp.full((L,), NEG, jnp.float32)

  def init(i, carry):
    for j in range(FPW):
      a[j][pl.ds(i * L, L)] = neg
    return carry

  lax.fori_loop(0, N // L, init, 0)

  iota = lax.iota(jnp.int32, L)
  tru = jnp.ones((L,), jnp.bool_)

  nchunks = EH // CHUNK

  def edma_start(ci, buf):
    esb, edb, ss, sd = buf
    base = half * EH + ci * CHUNK
    pltpu.make_async_copy(src_hbm.at[pl.ds(base, CHUNK)], esb, ss).start()
    pltpu.make_async_copy(dst_hbm.at[pl.ds(base, CHUNK)], edb, sd).start()

  def edma_wait(ci, buf):
    esb, edb, ss, sd = buf
    base = half * EH + ci * CHUNK
    pltpu.make_async_copy(src_hbm.at[pl.ds(base, CHUNK)], esb, ss).wait()
    pltpu.make_async_copy(dst_hbm.at[pl.ds(base, CHUNK)], edb, sd).wait()

  def chunk_work(ci, es, ed):

    def vec_one(k, viol, tmp_u):
      sv = es[pl.ds(k * L, L)]
      dv = ed[pl.ds(k * L, L)]
      vs = [plsc.load_gather(q[j], [sv]) for j in range(FPW)]
      plsc.store_scatter(tmp_u, [dv], iota)
      rd = plsc.load_gather(tmp_u, [dv])
      win = rd == iota
      cs = [plsc.load_gather(a[j], [dv]) for j in range(FPW)]
      ns = [jnp.maximum(cs[j], vs[j]) for j in range(FPW)]
      for j in range(FPW):
        plsc.store_scatter(a[j], [dv], ns[j], mask=win)
      pend = jnp.logical_not(win)
      plsc.store_scatter(tmp_u, [dv], iota, mask=pend)
      rd2 = plsc.load_gather(tmp_u, [dv], mask=pend)
      win2 = jnp.logical_and(rd2 == iota, pend)
      for j in range(FPW):
        pulled = jnp.take_along_axis(vs[j], rd, axis=0,
                                     mode="promise_in_bounds")
        plsc.store_scatter(a[j], [dv], jnp.maximum(ns[j], pulled), mask=win2)
      return jnp.logical_or(viol, jnp.logical_and(pend, jnp.logical_not(win2)))

    def vec_body(k2, viol):
      viol = vec_one(2 * k2, viol, tmp)
      viol = vec_one(2 * k2 + 1, viol, tmp2)
      return viol

    viol = lax.fori_loop(0, CHUNK // L // 2, vec_body,
                         jnp.zeros((L,), jnp.bool_))

    @pl.when(jnp.any(viol))
    def _slow_redo():
      def vec_slow(k, carry2):
        sv = es[pl.ds(k * L, L)]
        dv = ed[pl.ds(k * L, L)]
        vs = [plsc.load_gather(q[j], [sv]) for j in range(FPW)]

        def w_cond(state):
          return jnp.any(state[0])

        def w_body(state):
          pending = state[0]
          plsc.store_scatter(tmp, [dv], iota, mask=pending)
          rdw = plsc.load_gather(tmp, [dv], mask=pending)
          w_ = jnp.logical_and(rdw == iota, pending)
          for j in range(FPW):
            cj = plsc.load_gather(a[j], [dv], mask=w_)
            plsc.store_scatter(a[j], [dv], jnp.maximum(cj, vs[j]), mask=w_)
          return (jnp.logical_and(pending, jnp.logical_not(w_)),)

        lax.while_loop(w_cond, w_body, (tru,))
        return carry2

      lax.fori_loop(0, CHUNK // L, vec_slow, 0)

  def outer(ci2, carry):
    for par in range(2):
      ci = ci2 * 2 + par
      edma_wait(ci, ebufs[par])

      @pl.when(ci + 1 < nchunks)
      def _prefetch():
        edma_start(ci + 1, ebufs[1 - par])

      chunk_work(ci, ebufs[par][0], ebufs[par][1])
    return carry

  edma_start(0, ebufs[0])
  lax.fori_loop(0, nchunks // 2, outer, 0)

  for j in range(FPW):
    pltpu.sync_copy(a[j], out_hbm.at[half, f0 + j])


_segmax = functools.partial(
    pl.kernel,
    mesh=plsc.VectorSubcoreMesh(core_axis_name="c", subcore_axis_name="s"),
    out_type=jax.ShapeDtypeStruct((2, GR, N), jnp.float32),
    scratch_types=[
        pltpu.VMEM((N,), jnp.float32),
        pltpu.VMEM((N,), jnp.float32),
        pltpu.VMEM((N,), jnp.float32),
        pltpu.VMEM((N,), jnp.float32),
        pltpu.VMEM((N,), jnp.float32),
        pltpu.VMEM((N,), jnp.float32),
        pltpu.VMEM((N,), jnp.float32),
        pltpu.VMEM((N,), jnp.float32),
        pltpu.VMEM((CHUNK,), jnp.int32),
        pltpu.VMEM((CHUNK,), jnp.int32),
        pltpu.VMEM((CHUNK,), jnp.int32),
        pltpu.VMEM((CHUNK,), jnp.int32),
        pltpu.VMEM((L,), jnp.int32),
        pltpu.VMEM((L,), jnp.int32),
        pltpu.SemaphoreType.DMA,
        pltpu.SemaphoreType.DMA,
        pltpu.SemaphoreType.DMA,
        pltpu.SemaphoreType.DMA,
    ],
    compiler_params=pltpu.CompilerParams(needs_layout_passes=False),
)(_segmax_body)


def _tc0_body(xT, WlT, bl, Wc, bc, h0T_o, pq_o):
  h0 = jnp.dot(WlT[...], xT[...], preferred_element_type=jnp.float32) + bl[...]
  h0T_o[...] = h0
  pq_o[...] = jnp.dot(Wc[...], h0, preferred_element_type=jnp.float32) + bc[...]


_tc0 = pl.pallas_call(
    _tc0_body,
    out_shape=[
        jax.ShapeDtypeStruct((GR, N), jnp.float32),
        jax.ShapeDtypeStruct((2 * GR, N), jnp.float32),
    ],
)


def _tcb_body(nparts, pq, mT, Wc, bc, *refs):
  hrefs = refs[:nparts]
  agg_o, pq_o = refs[nparts], refs[nparts + 1]
  m = jnp.maximum(mT[0], mT[1])
  agg = jnp.where(m > -1.0e30, pq[0:GR, :] + m, 0.0)
  agg_o[...] = agg
  hcat = jnp.concatenate([h[...] for h in hrefs] + [agg], axis=0)
  pq_o[...] = jnp.dot(Wc[...], hcat, preferred_element_type=jnp.float32) + bc[...]


def _make_tcb(nparts):
  return pl.pallas_call(
      functools.partial(_tcb_body, nparts),
      out_shape=[
          jax.ShapeDtypeStruct((GR, N), jnp.float32),
          jax.ShapeDtypeStruct((2 * GR, N), jnp.float32),
      ],
  )


_tcb1 = _make_tcb(1)
_tcb2 = _make_tcb(2)


def _pool4(S):
  return jnp.max(S.reshape(GR // 4, 4, S.shape[-1]), axis=1)


def _tcf_body(h0T, a0T, a1T, pq, mT, out_o):
  m = jnp.maximum(mT[0], mT[1])
  a2 = jnp.where(m > -1.0e30, pq[0:GR, :] + m, 0.0)
  out_o[...] = jnp.concatenate(
      [_pool4(h0T[...]), _pool4(a0T[...]), _pool4(a1T[...]), _pool4(a2)],
      axis=0,
  )


_tcf = pl.pallas_call(
    _tcf_body,
    out_shape=jax.ShapeDtypeStruct((GR, N), jnp.float32),
)


def kernel(x, edge_index, lin_x_W, lin_x_b, W0, b0, W1, b1, W2, b2):
  xT = x.T
  src = edge_index[0]
  dst = edge_index[1]

  Wcs, bcs = [], []
  for i, (W, b) in enumerate(((W0, b0), (W1, b1), (W2, b2))):
    cin = (i + 1) * GR
    Wt = W[:cin].T
    Wb = W[cin:].T
    Wcs.append(jnp.concatenate([Wt - Wb, Wb], axis=0))
    bcs.append(jnp.concatenate([b, jnp.zeros((GR,), jnp.float32)])[:, None])

  h0T, pq = _tc0(xT, lin_x_W.T, lin_x_b[:, None], Wcs[0], bcs[0])
  m0 = _segmax(pq, src, dst)
  agg0, pq = _tcb1(pq, m0, Wcs[1], bcs[1], h0T)
  m1 = _segmax(pq, src, dst)
  agg1, pq = _tcb2(pq, m1, Wcs[2], bcs[2], h0T, agg0)
  m2 = _segmax(pq, src, dst)
  outT = _tcf(h0T, agg0, agg1, pq, m2)
  return outT.T


# speedup vs baseline: 3.0667x; 1.1111x over previous
import functools

import jax
import jax.numpy as jnp
from jax import lax
from jax.experimental import pallas as pl
from jax.experimental.pallas import tpu as pltpu
from jax.experimental.pallas import tpu_sc as plsc

N = 10000
E = 320000
GR = 64
D = 128
NEG = -3.0e38
CHUNK = 8000
L = 16
SHIFT = 14
FPW = 4
EH = E // 2


def _segmax_body(pq_hbm, ep_hbm, out_hbm, q0, q1, q2, q3, a0, a1,
                 a2, a3, es0, es1, tmp, tmp2, ss0, ss1):
  q = (q0, q1, q2, q3)
  a = (a0, a1, a2, a3)
  ebufs = ((es0, ss0), (es1, ss1))
  half = lax.axis_index("c")
  s = lax.axis_index("s")
  f0 = FPW * s

  for j in range(FPW):
    pltpu.sync_copy(pq_hbm.at[GR + f0 + j], q[j])

  neg = jnp.full((L,), NEG, jnp.float32)

  def init(i, carry):
    for j in range(FPW):
      a[j][pl.ds(i * L, L)] = neg
    return carry

  lax.fori_loop(0, N // L, init, 0)

  iota = lax.iota(jnp.int32, L)
  tru = jnp.ones((L,), jnp.bool_)

  nchunks = EH // CHUNK

  def edma_start(ci, buf):
    esb, ss = buf
    base = half * EH + ci * CHUNK
    pltpu.make_async_copy(ep_hbm.at[pl.ds(base, CHUNK)], esb, ss).start()

  def edma_wait(ci, buf):
    esb, ss = buf
    base = half * EH + ci * CHUNK
    pltpu.make_async_copy(ep_hbm.at[pl.ds(base, CHUNK)], esb, ss).wait()

  def chunk_work(ci, es):

    def vec_one(k, viol, tmp_u):
      epk = es[pl.ds(k * L, L)]
      dv = jnp.bitwise_and(epk, (1 << SHIFT) - 1)
      sv = jnp.right_shift(epk, SHIFT)
      plsc.store_scatter(tmp_u, [dv], iota)
      vs = [plsc.load_gather(q[j], [sv]) for j in range(FPW)]
      rd = plsc.load_gather(tmp_u, [dv])
      win = rd == iota
      pend = jnp.logical_not(win)
      plsc.store_scatter(tmp_u, [dv], iota, mask=pend)
      cs = [plsc.load_gather(a[j], [dv]) for j in range(FPW)]
      rd2 = plsc.load_gather(tmp_u, [dv], mask=pend)
      win2 = jnp.logical_and(rd2 == iota, pend)
      ns = [jnp.maximum(cs[j], vs[j]) for j in range(FPW)]
      for j in range(FPW):
        plsc.store_scatter(a[j], [dv], ns[j], mask=win)
      pulled = [jnp.take_along_axis(vs[j], rd, axis=0,
                                    mode="promise_in_bounds")
                for j in range(FPW)]
      for j in range(FPW):
        plsc.store_scatter(a[j], [dv], jnp.maximum(ns[j], pulled[j]),
                           mask=win2)
      return jnp.logical_or(viol, jnp.logical_and(pend, jnp.logical_not(win2)))

    def vec_body(k2, viol):
      viol = vec_one(2 * k2, viol, tmp)
      viol = vec_one(2 * k2 + 1, viol, tmp2)
      return viol

    viol = lax.fori_loop(0, CHUNK // L // 2, vec_body,
                         jnp.zeros((L,), jnp.bool_))

    @pl.when(jnp.any(viol))
    def _slow_redo():
      def vec_slow(k, carry2):
        epk = es[pl.ds(k * L, L)]
        dv = jnp.bitwise_and(epk, (1 << SHIFT) - 1)
        sv = jnp.right_shift(epk, SHIFT)
        vs = [plsc.load_gather(q[j], [sv]) for j in range(FPW)]

        def w_cond(state):
          return jnp.any(state[0])

        def w_body(state):
          pending = state[0]
          plsc.store_scatter(tmp, [dv], iota, mask=pending)
          rdw = plsc.load_gather(tmp, [dv], mask=pending)
          w_ = jnp.logical_and(rdw == iota, pending)
          for j in range(FPW):
            cj = plsc.load_gather(a[j], [dv], mask=w_)
            plsc.store_scatter(a[j], [dv], jnp.maximum(cj, vs[j]), mask=w_)
          return (jnp.logical_and(pending, jnp.logical_not(w_)),)

        lax.while_loop(w_cond, w_body, (tru,))
        return carry2

      lax.fori_loop(0, CHUNK // L, vec_slow, 0)

  def outer(ci2, carry):
    for par in range(2):
      ci = ci2 * 2 + par
      edma_wait(ci, ebufs[par])

      @pl.when(ci + 1 < nchunks)
      def _prefetch():
        edma_start(ci + 1, ebufs[1 - par])

      chunk_work(ci, ebufs[par][0])
    return carry

  edma_start(0, ebufs[0])
  lax.fori_loop(0, nchunks // 2, outer, 0)

  for j in range(FPW):
    pltpu.sync_copy(a[j], out_hbm.at[half, f0 + j])


_segmax = functools.partial(
    pl.kernel,
    mesh=plsc.VectorSubcoreMesh(core_axis_name="c", subcore_axis_name="s"),
    out_type=jax.ShapeDtypeStruct((2, GR, N), jnp.float32),
    scratch_types=[
        pltpu.VMEM((N,), jnp.float32),
        pltpu.VMEM((N,), jnp.float32),
        pltpu.VMEM((N,), jnp.float32),
        pltpu.VMEM((N,), jnp.float32),
        pltpu.VMEM((N,), jnp.float32),
        pltpu.VMEM((N,), jnp.float32),
        pltpu.VMEM((N,), jnp.float32),
        pltpu.VMEM((N,), jnp.float32),
        pltpu.VMEM((CHUNK,), jnp.int32),
        pltpu.VMEM((CHUNK,), jnp.int32),
        pltpu.VMEM((L,), jnp.int32),
        pltpu.VMEM((L,), jnp.int32),
        pltpu.SemaphoreType.DMA,
        pltpu.SemaphoreType.DMA,
    ],
    compiler_params=pltpu.CompilerParams(needs_layout_passes=False),
)(_segmax_body)


def _pack_body(ei, out_o):
  out_o[...] = jnp.bitwise_or(ei[1, :], jnp.left_shift(ei[0, :], SHIFT))


_pack = pl.pallas_call(
    _pack_body,
    out_shape=jax.ShapeDtypeStruct((E,), jnp.int32),
)


def _tc0_body(xT, WlT, bl, Wc, bc, h0T_o, pq_o):
  h0 = jnp.dot(WlT[...], xT[...], preferred_element_type=jnp.float32) + bl[...]
  h0T_o[...] = h0
  pq_o[...] = jnp.dot(Wc[...], h0, preferred_element_type=jnp.float32) + bc[...]


_tc0 = pl.pallas_call(
    _tc0_body,
    out_shape=[
        jax.ShapeDtypeStruct((GR, N), jnp.float32),
        jax.ShapeDtypeStruct((2 * GR, N), jnp.float32),
    ],
)


def _tcb_body(nparts, pq, mT, Wc, bc, *refs):
  hrefs = refs[:nparts]
  agg_o, pq_o = refs[nparts], refs[nparts + 1]
  m = jnp.maximum(mT[0], mT[1])
  agg = jnp.where(m > -1.0e30, pq[0:GR, :] + m, 0.0)
  agg_o[...] = agg
  hcat = jnp.concatenate([h[...] for h in hrefs] + [agg], axis=0)
  pq_o[...] = jnp.dot(Wc[...], hcat, preferred_element_type=jnp.float32) + bc[...]


def _make_tcb(nparts):
  return pl.pallas_call(
      functools.partial(_tcb_body, nparts),
      out_shape=[
          jax.ShapeDtypeStruct((GR, N), jnp.float32),
          jax.ShapeDtypeStruct((2 * GR, N), jnp.float32),
      ],
  )


_tcb1 = _make_tcb(1)
_tcb2 = _make_tcb(2)


def _pool4(S):
  return jnp.max(S.reshape(GR // 4, 4, S.shape[-1]), axis=1)


def _tcf_body(h0T, a0T, a1T, pq, mT, out_o):
  m = jnp.maximum(mT[0], mT[1])
  a2 = jnp.where(m > -1.0e30, pq[0:GR, :] + m, 0.0)
  out_o[...] = jnp.concatenate(
      [_pool4(h0T[...]), _pool4(a0T[...]), _pool4(a1T[...]), _pool4(a2)],
      axis=0,
  )


_tcf = pl.pallas_call(
    _tcf_body,
    out_shape=jax.ShapeDtypeStruct((GR, N), jnp.float32),
)


def kernel(x, edge_index, lin_x_W, lin_x_b, W0, b0, W1, b1, W2, b2):
  xT = x.T
  ep = _pack(edge_index)

  Wcs, bcs = [], []
  for i, (W, b) in enumerate(((W0, b0), (W1, b1), (W2, b2))):
    cin = (i + 1) * GR
    Wt = W[:cin].T
    Wb = W[cin:].T
    Wcs.append(jnp.concatenate([Wt - Wb, Wb], axis=0))
    bcs.append(jnp.concatenate([b, jnp.zeros((GR,), jnp.float32)])[:, None])

  h0T, pq = _tc0(xT, lin_x_W.T, lin_x_b[:, None], Wcs[0], bcs[0])
  m0 = _segmax(pq, ep)
  agg0, pq = _tcb1(pq, m0, Wcs[1], bcs[1], h0T)
  m1 = _segmax(pq, ep)
  agg1, pq = _tcb2(pq, m1, Wcs[2], bcs[2], h0T, agg0)
  m2 = _segmax(pq, ep)
  outT = _tcf(h0T, agg0, agg1, pq, m2)
  return outT.T
